# Initial kernel scaffold; baseline (speedup 1.0000x reference)
import jax
import jax.numpy as jnp
from jax.experimental import pallas as pl


def kernel(indices, bin_values):
    raise NotImplementedError("write your pallas kernel here")


# speedup vs baseline: 194.2403x; 194.2403x over previous
import functools

import jax
import jax.numpy as jnp
from jax import lax
from jax.experimental import pallas as pl
from jax.experimental.pallas import tpu as pltpu
from jax.experimental.pallas import tpu_sc as plsc

NUM_BINS = 100000
L = 16
NC = 2
NS = 16
NW = NC * NS
CHUNK = 4096


def _sc_lookup(total, per_w, n_chunks):
    mesh = plsc.VectorSubcoreMesh(core_axis_name="c", subcore_axis_name="s")

    @functools.partial(
        pl.kernel,
        mesh=mesh,
        out_type=jax.ShapeDtypeStruct((total,), jnp.float32),
        scratch_types=[
            pltpu.VMEM((NUM_BINS,), jnp.float32),
            pltpu.VMEM((CHUNK,), jnp.int32),
            pltpu.VMEM((CHUNK,), jnp.float32),
            pltpu.SemaphoreType.DMA,
        ],
        compiler_params=pltpu.CompilerParams(needs_layout_passes=False),
    )
    def sc_kernel(idx_hbm, table_hbm, out_hbm, table_v, idx_v, out_v, sem):
        del sem
        wid = lax.axis_index("s") * NC + lax.axis_index("c")
        base = wid * per_w
        pltpu.sync_copy(table_hbm, table_v)

        def chunk_body(ci, carry):
            off = base + ci * CHUNK
            pltpu.sync_copy(idx_hbm.at[pl.ds(off, CHUNK)], idx_v)

            def gather_body(j, c):
                idx = idx_v[pl.ds(j * L, L)]
                idx = jnp.minimum(jnp.maximum(idx, 0), NUM_BINS - 1)
                out_v[pl.ds(j * L, L)] = plsc.load_gather(table_v, [idx])
                return c

            lax.fori_loop(0, CHUNK // L, gather_body, 0)
            pltpu.sync_copy(out_v, out_hbm.at[pl.ds(off, CHUNK)])
            return carry

        lax.fori_loop(0, n_chunks, chunk_body, 0)

    return sc_kernel


def kernel(indices, bin_values):
    b, h = indices.shape
    total = b * h
    per_w = total // NW
    n_chunks = per_w // CHUNK
    flat_idx = indices.reshape(total).astype(jnp.int32)
    out = _sc_lookup(total, per_w, n_chunks)(flat_idx, bin_values)
    return out.reshape(b, h)


# speedup vs baseline: 217.6405x; 1.1205x over previous
import functools

import jax
import jax.numpy as jnp
from jax import lax
from jax.experimental import pallas as pl
from jax.experimental.pallas import tpu as pltpu
from jax.experimental.pallas import tpu_sc as plsc

NUM_BINS = 100000
L = 16
NC = 2
NS = 16
NW = NC * NS
CHUNK = 4096


def _sc_lookup(total, per_w, n_chunks):
    mesh = plsc.VectorSubcoreMesh(core_axis_name="c", subcore_axis_name="s")

    @functools.partial(
        pl.kernel,
        mesh=mesh,
        out_type=jax.ShapeDtypeStruct((total,), jnp.float32),
        scratch_types=[
            pltpu.VMEM((NUM_BINS,), jnp.float32),
            pltpu.VMEM((CHUNK,), jnp.int32),
            pltpu.VMEM((CHUNK,), jnp.float32),
            pltpu.SemaphoreType.DMA,
        ],
        compiler_params=pltpu.CompilerParams(needs_layout_passes=False),
    )
    def sc_kernel(idx_hbm, table_hbm, out_hbm, table_v, idx_v, out_v, sem):
        del sem
        wid = lax.axis_index("s") * NC + lax.axis_index("c")
        base = wid * per_w
        pltpu.sync_copy(table_hbm, table_v)

        def chunk_body(ci, carry):
            off = base + ci * CHUNK
            pltpu.sync_copy(idx_hbm.at[pl.ds(off, CHUNK)], idx_v)

            @plsc.parallel_loop(0, CHUNK, step=L, unroll=8)
            def gather_body(j):
                idx = idx_v[pl.ds(j, L)]
                idx = jnp.minimum(jnp.maximum(idx, 0), NUM_BINS - 1)
                out_v[pl.ds(j, L)] = plsc.load_gather(table_v, [idx])
            pltpu.sync_copy(out_v, out_hbm.at[pl.ds(off, CHUNK)])
            return carry

        lax.fori_loop(0, n_chunks, chunk_body, 0)

    return sc_kernel


def kernel(indices, bin_values):
    b, h = indices.shape
    total = b * h
    per_w = total // NW
    n_chunks = per_w // CHUNK
    flat_idx = indices.reshape(total).astype(jnp.int32)
    out = _sc_lookup(total, per_w, n_chunks)(flat_idx, bin_values)
    return out.reshape(b, h)


# speedup vs baseline: 249.9337x; 1.1484x over previous
import functools

import jax
import jax.numpy as jnp
from jax import lax
from jax.experimental import pallas as pl
from jax.experimental.pallas import tpu as pltpu
from jax.experimental.pallas import tpu_sc as plsc

NUM_BINS = 100000
L = 16
NC = 2
NS = 16
NW = NC * NS
CHUNK = 6400
NBUF = 2


def _sc_lookup(total, per_w, n_chunks):
    mesh = plsc.VectorSubcoreMesh(core_axis_name="c", subcore_axis_name="s")

    @functools.partial(
        pl.kernel,
        mesh=mesh,
        out_type=jax.ShapeDtypeStruct((total,), jnp.float32),
        scratch_types=[
            pltpu.VMEM((NUM_BINS,), jnp.float32),
            pltpu.VMEM((NBUF, CHUNK), jnp.int32),
            pltpu.VMEM((NBUF, CHUNK), jnp.float32),
            pltpu.SemaphoreType.DMA((NBUF,)),
            pltpu.SemaphoreType.DMA((NBUF,)),
        ],
        compiler_params=pltpu.CompilerParams(needs_layout_passes=False),
    )
    def sc_kernel(idx_hbm, table_hbm, out_hbm, table_v, idx_v, out_v,
                  sem_in, sem_out):
        wid = lax.axis_index("s") * NC + lax.axis_index("c")
        base = wid * per_w

        def in_copy(ci, b):
            return pltpu.make_async_copy(
                idx_hbm.at[pl.ds(base + ci * CHUNK, CHUNK)],
                idx_v.at[b], sem_in.at[b])

        def out_copy(ci, b):
            return pltpu.make_async_copy(
                out_v.at[b],
                out_hbm.at[pl.ds(base + ci * CHUNK, CHUNK)], sem_out.at[b])

        for b in range(NBUF):
            in_copy(b, b).start()
        pltpu.sync_copy(table_hbm, table_v)

        @pl.loop(0, n_chunks, step=NBUF)
        def chunk_pair(c0):
            for b in range(NBUF):
                ci = c0 + b
                in_copy(ci, b).wait()

                @pl.when(ci >= NBUF)
                def _wait_out(b=b, ci=ci):
                    out_copy(ci - NBUF, b).wait()

                @plsc.parallel_loop(0, CHUNK, step=L, unroll=8)
                def gather_body(j, b=b):
                    idx = idx_v[b, pl.ds(j, L)]
                    idx = jnp.minimum(jnp.maximum(idx, 0), NUM_BINS - 1)
                    out_v[b, pl.ds(j, L)] = plsc.load_gather(table_v, [idx])

                out_copy(ci, b).start()

                @pl.when(ci + NBUF < n_chunks)
                def _next_in(b=b, ci=ci):
                    in_copy(ci + NBUF, b).start()

        for b in range(NBUF):
            out_copy(n_chunks - NBUF + b, b).wait()

    return sc_kernel


def kernel(indices, bin_values):
    b, h = indices.shape
    total = b * h
    per_w = total // NW
    n_chunks = per_w // CHUNK
    flat_idx = indices.reshape(total).astype(jnp.int32)
    out = _sc_lookup(total, per_w, n_chunks)(flat_idx, bin_values)
    return out.reshape(b, h)


# speedup vs baseline: 393.9134x; 1.5761x over previous
import functools

import jax
import jax.numpy as jnp
from jax import lax
from jax.experimental import pallas as pl
from jax.experimental.pallas import tpu as pltpu
from jax.experimental.pallas import tpu_sc as plsc

NUM_BINS = 100000
L = 16
NC = 2
NS = 16
NW = NC * NS
RCH = 16
NBUF = 2


def _sc_lookup(n_rows, n_cols):
    rows_per_w = n_rows // NW
    n_chunks = rows_per_w // RCH
    n_full = n_cols // L
    rem = n_cols - n_full * L

    mesh = plsc.VectorSubcoreMesh(core_axis_name="c", subcore_axis_name="s")

    @functools.partial(
        pl.kernel,
        mesh=mesh,
        out_type=jax.ShapeDtypeStruct((n_rows, n_cols), jnp.float32),
        scratch_types=[
            pltpu.VMEM((NUM_BINS,), jnp.float32),
            pltpu.VMEM((RCH, n_cols), jnp.int32),
            pltpu.VMEM((RCH, n_cols), jnp.int32),
            pltpu.VMEM((RCH, n_cols), jnp.float32),
            pltpu.VMEM((RCH, n_cols), jnp.float32),
            pltpu.SemaphoreType.DMA((NBUF,)),
            pltpu.SemaphoreType.DMA((NBUF,)),
        ],
        compiler_params=pltpu.CompilerParams(needs_layout_passes=False),
    )
    def sc_kernel(idx_hbm, table_hbm, out_hbm, table_v,
                  idx_v0, idx_v1, out_v0, out_v1, sem_in, sem_out):
        wid = lax.axis_index("s") * NC + lax.axis_index("c")
        rbase = wid * rows_per_w
        idx_bufs = (idx_v0, idx_v1)
        out_bufs = (out_v0, out_v1)

        def in_copy(i, b):
            return pltpu.make_async_copy(
                idx_hbm.at[pl.ds(rbase + i * RCH, RCH), :],
                idx_bufs[b], sem_in.at[b])

        def out_copy(i, b):
            return pltpu.make_async_copy(
                out_bufs[b],
                out_hbm.at[pl.ds(rbase + i * RCH, RCH), :], sem_out.at[b])

        for b in range(NBUF):
            in_copy(b, b).start()
        pltpu.sync_copy(table_hbm, table_v)

        def clamped_gather(idx):
            idx = jnp.minimum(jnp.maximum(idx, 0), NUM_BINS - 1)
            return plsc.load_gather(table_v, [idx])

        iota = lax.iota(jnp.int32, L)
        hi = (iota >= rem).astype(jnp.int32) if rem else None
        cols = n_full * L + (iota - rem * hi) if rem else None

        @pl.loop(0, n_chunks, step=NBUF)
        def _chunks(c0):
            for b in range(NBUF):
                i = c0 + b
                in_copy(i, b).wait()

                @pl.when(i >= NBUF)
                def _wait_out(i=i, b=b):
                    out_copy(i - NBUF, b).wait()

                @plsc.parallel_loop(0, RCH, unroll=2)
                def _gather_row(r, b=b):
                    for c in range(n_full):
                        idx = idx_bufs[b][r, pl.ds(c * L, L)]
                        out_bufs[b][r, pl.ds(c * L, L)] = clamped_gather(idx)

                if rem:
                    @plsc.parallel_loop(0, RCH // 2, unroll=4)
                    def _gather_tail(v, b=b):
                        rows = 2 * v + hi
                        idx = plsc.load_gather(idx_bufs[b], [rows, cols])
                        vals = clamped_gather(idx)
                        plsc.store_scatter(out_bufs[b], [rows, cols], vals)

                out_copy(i, b).start()

                @pl.when(i + NBUF < n_chunks)
                def _next_in(i=i, b=b):
                    in_copy(i + NBUF, b).start()

        for b in range(NBUF):
            out_copy(n_chunks - NBUF + b, b).wait()

    return sc_kernel


def kernel(indices, bin_values):
    n_rows, n_cols = indices.shape
    return _sc_lookup(n_rows, n_cols)(indices.astype(jnp.int32), bin_values)


# speedup vs baseline: 657.9115x; 1.6702x over previous
import functools

import jax
import jax.numpy as jnp
from jax import lax
from jax.experimental import pallas as pl
from jax.experimental.pallas import tpu as pltpu
from jax.experimental.pallas import tpu_sc as plsc

NUM_BINS = 100000
L = 16
NC = 2
NS = 16
NW = NC * NS
RCH = 8
NBUF = 2


def _sc_lookup(n_rows, n_cols):
    cols_per_w = n_cols // NW
    n_chunks = n_rows // RCH
    vecs = RCH * cols_per_w // L

    mesh = plsc.VectorSubcoreMesh(core_axis_name="c", subcore_axis_name="s")

    @functools.partial(
        pl.kernel,
        mesh=mesh,
        out_type=jax.ShapeDtypeStruct((n_rows, n_cols), jnp.float32),
        scratch_types=[
            pltpu.VMEM((NUM_BINS,), jnp.float32),
            pltpu.VMEM((RCH, cols_per_w), jnp.int32),
            pltpu.VMEM((RCH, cols_per_w), jnp.int32),
            pltpu.VMEM((RCH, cols_per_w), jnp.float32),
            pltpu.VMEM((RCH, cols_per_w), jnp.float32),
            pltpu.SemaphoreType.DMA((NBUF,)),
            pltpu.SemaphoreType.DMA((NBUF,)),
        ],
        compiler_params=pltpu.CompilerParams(needs_layout_passes=False),
    )
    def sc_kernel(idx_hbm, table_hbm, out_hbm, table_v,
                  idx_v0, idx_v1, out_v0, out_v1, sem_in, sem_out):
        wid = lax.axis_index("s") * NC + lax.axis_index("c")
        cbase = wid * cols_per_w
        idx_bufs = (idx_v0, idx_v1)
        out_bufs = (out_v0, out_v1)
        cshift = (cols_per_w // L).bit_length() - 1

        def in_copy(i, b):
            return pltpu.make_async_copy(
                idx_hbm.at[pl.ds(i * RCH, RCH), pl.ds(cbase, cols_per_w)],
                idx_bufs[b], sem_in.at[b])

        def out_copy(i, b):
            return pltpu.make_async_copy(
                out_bufs[b],
                out_hbm.at[pl.ds(i * RCH, RCH), pl.ds(cbase, cols_per_w)],
                sem_out.at[b])

        for b in range(NBUF):
            in_copy(b, b).start()
        pltpu.sync_copy(table_hbm, table_v)

        def _maybe(cond, fn):
            if isinstance(cond, bool):
                if cond:
                    fn()
            else:
                pl.when(cond)(fn)

        def process(i, b):
            in_copy(i, b).wait()
            _maybe(i >= NBUF, lambda: out_copy(i - NBUF, b).wait())

            @plsc.parallel_loop(0, vecs, unroll=8)
            def _gather(t):
                r = lax.shift_right_logical(t, cshift)
                c = lax.shift_left(t & ((1 << cshift) - 1), 4)
                idx = idx_bufs[b][r, pl.ds(c, L)]
                idx = jnp.minimum(jnp.maximum(idx, 0), NUM_BINS - 1)
                out_bufs[b][r, pl.ds(c, L)] = plsc.load_gather(table_v, [idx])

            out_copy(i, b).start()
            _maybe(i + NBUF < n_chunks, lambda: in_copy(i + NBUF, b).start())

        n_paired = n_chunks - (n_chunks % NBUF)

        @pl.loop(0, n_paired, step=NBUF)
        def _chunks(c0):
            for b in range(NBUF):
                process(c0 + b, b)

        for i in range(n_paired, n_chunks):
            process(i, i % NBUF)

        for i in range(n_chunks - NBUF, n_chunks):
            out_copy(i, i % NBUF).wait()

    return sc_kernel


def kernel(indices, bin_values):
    n_rows, n_cols = indices.shape
    out_t = _sc_lookup(n_cols, n_rows)(indices.astype(jnp.int32).T, bin_values)
    return out_t.T


# speedup vs baseline: 722.0404x; 1.0975x over previous
import functools

import jax
import jax.numpy as jnp
from jax import lax
from jax.experimental import pallas as pl
from jax.experimental.pallas import tpu as pltpu
from jax.experimental.pallas import tpu_sc as plsc

NUM_BINS = 100000
L = 16
NC = 2
NS = 16
NW = NC * NS
RCH = 8
NBUF = 3


def _sc_lookup(n_rows, n_cols):
    cols_per_w = n_cols // NW
    n_chunks = n_rows // RCH
    vecs = RCH * cols_per_w // L

    mesh = plsc.VectorSubcoreMesh(core_axis_name="c", subcore_axis_name="s")

    @functools.partial(
        pl.kernel,
        mesh=mesh,
        out_type=jax.ShapeDtypeStruct((n_rows, n_cols), jnp.float32),
        scratch_types=(
            [pltpu.VMEM((NUM_BINS,), jnp.float32)]
            + [pltpu.VMEM((RCH, cols_per_w), jnp.int32) for _ in range(NBUF)]
            + [pltpu.VMEM((RCH, cols_per_w), jnp.float32) for _ in range(NBUF)]
            + [pltpu.SemaphoreType.DMA((NBUF,)),
               pltpu.SemaphoreType.DMA((NBUF,))]
        ),
        compiler_params=pltpu.CompilerParams(needs_layout_passes=False),
    )
    def sc_kernel(idx_hbm, table_hbm, out_hbm, table_v, *rest):
        idx_bufs = rest[:NBUF]
        out_bufs = rest[NBUF:2 * NBUF]
        sem_in, sem_out = rest[2 * NBUF], rest[2 * NBUF + 1]
        wid = lax.axis_index("s") * NC + lax.axis_index("c")
        cbase = wid * cols_per_w
        cshift = (cols_per_w // L).bit_length() - 1

        def in_copy(i, b):
            return pltpu.make_async_copy(
                idx_hbm.at[pl.ds(i * RCH, RCH), pl.ds(cbase, cols_per_w)],
                idx_bufs[b], sem_in.at[b])

        def out_copy(i, b):
            return pltpu.make_async_copy(
                out_bufs[b],
                out_hbm.at[pl.ds(i * RCH, RCH), pl.ds(cbase, cols_per_w)],
                sem_out.at[b])

        for b in range(NBUF):
            in_copy(b, b).start()
        pltpu.sync_copy(table_hbm, table_v)

        def _maybe(cond, fn):
            if isinstance(cond, bool):
                if cond:
                    fn()
            else:
                pl.when(cond)(fn)

        def process(i, b):
            in_copy(i, b).wait()
            _maybe(i >= NBUF, lambda: out_copy(i - NBUF, b).wait())

            @plsc.parallel_loop(0, vecs, unroll=16)
            def _gather(t):
                r = lax.shift_right_logical(t, cshift)
                c = lax.shift_left(t & ((1 << cshift) - 1), 4)
                idx = idx_bufs[b][r, pl.ds(c, L)]
                idx = jnp.minimum(jnp.maximum(idx, 0), NUM_BINS - 1)
                out_bufs[b][r, pl.ds(c, L)] = plsc.load_gather(table_v, [idx])

            out_copy(i, b).start()
            _maybe(i + NBUF < n_chunks, lambda: in_copy(i + NBUF, b).start())

        n_paired = n_chunks - (n_chunks % NBUF)

        @pl.loop(0, n_paired, step=NBUF)
        def _chunks(c0):
            for b in range(NBUF):
                process(c0 + b, b)

        for i in range(n_paired, n_chunks):
            process(i, i % NBUF)

        for i in range(n_chunks - NBUF, n_chunks):
            out_copy(i, i % NBUF).wait()

    return sc_kernel


def kernel(indices, bin_values):
    n_rows, n_cols = indices.shape
    out_t = _sc_lookup(n_cols, n_rows)(indices.astype(jnp.int32).T, bin_values)
    return out_t.T
